# Initial kernel scaffold; baseline (speedup 1.0000x reference)
import jax
import jax.numpy as jnp
from jax.experimental import pallas as pl


def kernel(anchors, objectness, box_regression):
    raise NotImplementedError("write your pallas kernel here")


# speedup vs baseline: 2.8340x; 2.8340x over previous
import functools
import math

import jax
import jax.numpy as jnp
from jax.experimental import pallas as pl

_NIMG = 4
_A = 3
_H = 192
_W = 256
_PRE = 2000
_POST = 1000
_NMS_T = 0.7
_MIN_SIZE = 0.0
_IM_W = 1024.0
_IM_H = 800.0
_CLIP = math.log(1000.0 / 16.0)
_NA = _A * _H * _W


def _nms_body(s_ref, anc_ref, breg_ref, sm_ref, box_ref):
    anc = anc_ref[0]
    rel = breg_ref[0]
    scores = s_ref[0]

    ax1 = anc[0:1, :]
    ay1 = anc[1:2, :]
    ax2 = anc[2:3, :]
    ay2 = anc[3:4, :]
    aw = ax2 - ax1 + 1.0
    ah = ay2 - ay1 + 1.0
    acx = ax1 + 0.5 * aw
    acy = ay1 + 0.5 * ah

    dx = rel[0:1, :]
    dy = rel[1:2, :]
    dw = jnp.minimum(rel[2:3, :], _CLIP)
    dh = jnp.minimum(rel[3:4, :], _CLIP)

    pcx = dx * aw + acx
    pcy = dy * ah + acy
    pw = jnp.exp(dw) * aw
    ph = jnp.exp(dh) * ah

    x1 = jnp.clip(pcx - 0.5 * pw, 0.0, _IM_W - 1.0)
    y1 = jnp.clip(pcy - 0.5 * ph, 0.0, _IM_H - 1.0)
    x2 = jnp.clip(pcx + 0.5 * pw - 1.0, 0.0, _IM_W - 1.0)
    y2 = jnp.clip(pcy + 0.5 * ph - 1.0, 0.0, _IM_H - 1.0)

    ws = x2 - x1 + 1.0
    hs = y2 - y1 + 1.0
    small_ok = (ws >= _MIN_SIZE) & (hs >= _MIN_SIZE)
    area = ws * hs

    iota = jax.lax.broadcasted_iota(jnp.int32, (1, _PRE), 1)

    def body(i, keep):
        sel = iota == i
        kx1 = jnp.max(jnp.where(sel, x1, -1e30), keepdims=True)
        ky1 = jnp.max(jnp.where(sel, y1, -1e30), keepdims=True)
        kx2 = jnp.max(jnp.where(sel, x2, -1e30), keepdims=True)
        ky2 = jnp.max(jnp.where(sel, y2, -1e30), keepdims=True)
        kar = jnp.max(jnp.where(sel, area, -1e30), keepdims=True)
        klive = jnp.max(jnp.where(sel, keep, 0.0), keepdims=True)
        iw = jnp.maximum(jnp.minimum(x2, kx2) - jnp.maximum(x1, kx1) + 1.0, 0.0)
        ih = jnp.maximum(jnp.minimum(y2, ky2) - jnp.maximum(y1, ky1) + 1.0, 0.0)
        inter = iw * ih
        iou = inter / (area + kar - inter)
        sup = (iou > _NMS_T) & (iota > i) & (klive > 0.5)
        return keep * jnp.where(sup, 0.0, 1.0)

    keep = jax.lax.fori_loop(0, _PRE, body, jnp.ones((1, _PRE), jnp.float32))

    sm_ref[0] = jnp.where((keep > 0.5) & small_ok, scores, -1e9)
    box_ref[0] = jnp.concatenate([x1, y1, x2, y2], axis=0)


def _run_nms(scores3, anc3, breg3):
    return pl.pallas_call(
        _nms_body,
        grid=(_NIMG,),
        in_specs=[
            pl.BlockSpec((1, 1, _PRE), lambda i: (i, 0, 0)),
            pl.BlockSpec((1, 4, _PRE), lambda i: (i, 0, 0)),
            pl.BlockSpec((1, 4, _PRE), lambda i: (i, 0, 0)),
        ],
        out_specs=[
            pl.BlockSpec((1, 1, _PRE), lambda i: (i, 0, 0)),
            pl.BlockSpec((1, 4, _PRE), lambda i: (i, 0, 0)),
        ],
        out_shape=[
            jax.ShapeDtypeStruct((_NIMG, 1, _PRE), jnp.float32),
            jax.ShapeDtypeStruct((_NIMG, 4, _PRE), jnp.float32),
        ],
    )(scores3, anc3, breg3)


@jax.jit
def kernel(anchors, objectness, box_regression):
    n = objectness.shape[0]
    obj = jnp.transpose(objectness, (0, 2, 3, 1)).reshape(n, -1)
    obj = jax.nn.sigmoid(obj)
    breg = (
        box_regression.reshape(n, _A, 4, _H, _W)
        .transpose(0, 3, 4, 1, 2)
        .reshape(n, -1, 4)
    )
    topk_scores, topk_idx = jax.lax.top_k(obj, _PRE)
    breg_sel = jnp.take_along_axis(breg, topk_idx[:, :, None], axis=1)
    anc_sel = jnp.take_along_axis(anchors, topk_idx[:, :, None], axis=1)

    scores3 = topk_scores.reshape(n, 1, _PRE)
    anc3 = jnp.transpose(anc_sel, (0, 2, 1))
    breg3 = jnp.transpose(breg_sel, (0, 2, 1))

    sm, boxes_t = _run_nms(scores3, anc3, breg3)
    sm = sm.reshape(n, _PRE)
    boxes = jnp.transpose(boxes_t, (0, 2, 1))

    top_scores, top_idx = jax.lax.top_k(sm, _POST)
    out_boxes = jnp.take_along_axis(boxes, top_idx[:, :, None], axis=1)
    valid = top_scores > -1e8
    out_boxes = jnp.where(valid[:, :, None], out_boxes, 0.0)
    out_scores = jnp.where(valid, top_scores, 0.0)
    return out_boxes, out_scores


# speedup vs baseline: 3.3121x; 1.1687x over previous
import functools
import math

import jax
import jax.numpy as jnp
from jax.experimental import pallas as pl

_NIMG = 4
_A = 3
_H = 192
_W = 256
_PRE = 2000
_PAD = 2048
_CHUNK = 128
_NCHUNK = _PAD // _CHUNK
_POST = 1000
_NMS_T = 0.7
_MIN_SIZE = 0.0
_IM_W = 1024.0
_IM_H = 800.0
_CLIP = math.log(1000.0 / 16.0)


def _decode(anc, rel):
    ax1, ay1, ax2, ay2 = anc[0], anc[1], anc[2], anc[3]
    aw = ax2 - ax1 + 1.0
    ah = ay2 - ay1 + 1.0
    acx = ax1 + 0.5 * aw
    acy = ay1 + 0.5 * ah
    dx = rel[0]
    dy = rel[1]
    dw = jnp.minimum(rel[2], _CLIP)
    dh = jnp.minimum(rel[3], _CLIP)
    pcx = dx * aw + acx
    pcy = dy * ah + acy
    pw = jnp.exp(dw) * aw
    ph = jnp.exp(dh) * ah
    x1 = jnp.clip(pcx - 0.5 * pw, 0.0, _IM_W - 1.0)
    y1 = jnp.clip(pcy - 0.5 * ph, 0.0, _IM_H - 1.0)
    x2 = jnp.clip(pcx + 0.5 * pw - 1.0, 0.0, _IM_W - 1.0)
    y2 = jnp.clip(pcy + 0.5 * ph - 1.0, 0.0, _IM_H - 1.0)
    return x1, y1, x2, y2


def _nms_body(s_ref, anc_ref, breg_ref, anct_ref, bregt_ref, sm_ref, box_ref):
    scores = s_ref[0]
    anc = anc_ref[0]
    rel = breg_ref[0]
    x1, y1, x2, y2 = _decode(
        [anc[i : i + 1, :] for i in range(4)],
        [rel[i : i + 1, :] for i in range(4)],
    )
    anct = anct_ref[0]
    relt = bregt_ref[0]
    x1t, y1t, x2t, y2t = _decode(
        [anct[i] for i in range(4)],
        [relt[i] for i in range(4)],
    )

    ws = x2 - x1 + 1.0
    hs = y2 - y1 + 1.0
    small_ok = (ws >= _MIN_SIZE) & (hs >= _MIN_SIZE)
    area = ws * hs
    wst = x2t - x1t + 1.0
    hst = y2t - y1t + 1.0
    areat = wst * hst

    lane128 = jax.lax.broadcasted_iota(jnp.int32, (1, _CHUNK), 1)
    glob = jax.lax.broadcasted_iota(jnp.int32, (1, _PAD), 1)
    tmat = (
        jax.lax.broadcasted_iota(jnp.int32, (_CHUNK, _CHUNK), 0)
        == jax.lax.broadcasted_iota(jnp.int32, (_CHUNK, _CHUNK), 1)
    )

    keep = jnp.ones((1, _PAD), jnp.float32)

    for c in range(_NCHUNK):
        lo = c * _CHUNK
        hi = lo + _CHUNK
        cx1 = x1[:, lo:hi]
        cy1 = y1[:, lo:hi]
        cx2 = x2[:, lo:hi]
        cy2 = y2[:, lo:hi]
        car = area[:, lo:hi]
        kchunk = keep[:, lo:hi]

        def body(k, kc, cx1=cx1, cy1=cy1, cx2=cx2, cy2=cy2, car=car):
            sel = lane128 == k
            kx1 = jnp.max(jnp.where(sel, cx1, -1e30), keepdims=True)
            ky1 = jnp.max(jnp.where(sel, cy1, -1e30), keepdims=True)
            kx2 = jnp.max(jnp.where(sel, cx2, -1e30), keepdims=True)
            ky2 = jnp.max(jnp.where(sel, cy2, -1e30), keepdims=True)
            kar = jnp.max(jnp.where(sel, car, -1e30), keepdims=True)
            klive = jnp.max(jnp.where(sel, kc, 0.0), keepdims=True)
            iw = jnp.maximum(jnp.minimum(cx2, kx2) - jnp.maximum(cx1, kx1) + 1.0, 0.0)
            ih = jnp.maximum(jnp.minimum(cy2, ky2) - jnp.maximum(cy1, ky1) + 1.0, 0.0)
            inter = iw * ih
            iou = inter / (car + kar - inter)
            sup = (iou > _NMS_T) & (lane128 > k) & (klive > 0.5)
            return kc * jnp.where(sup, 0.0, 1.0)

        kchunk = jax.lax.fori_loop(0, _CHUNK, body, kchunk)

        kt = jnp.max(
            jnp.where(tmat & (kchunk > 0.5), 1.0, 0.0), axis=1, keepdims=True
        )
        ccx1 = x1t[:, c : c + 1]
        ccy1 = y1t[:, c : c + 1]
        ccx2 = x2t[:, c : c + 1]
        ccy2 = y2t[:, c : c + 1]
        ccar = areat[:, c : c + 1]

        iw = jnp.maximum(jnp.minimum(x2, ccx2) - jnp.maximum(x1, ccx1) + 1.0, 0.0)
        ih = jnp.maximum(jnp.minimum(y2, ccy2) - jnp.maximum(y1, ccy1) + 1.0, 0.0)
        inter = iw * ih
        iou = inter / (area + ccar - inter)
        supm = (iou > _NMS_T) & (glob >= hi) & (kt > 0.5)
        sup_any = jnp.max(jnp.where(supm, 1.0, 0.0), axis=0, keepdims=True)

        parts = []
        if lo > 0:
            parts.append(keep[:, :lo])
        parts.append(kchunk)
        if hi < _PAD:
            parts.append(keep[:, hi:])
        keep = jnp.concatenate(parts, axis=1) * jnp.where(sup_any > 0.5, 0.0, 1.0)

    sm_ref[0] = jnp.where((keep > 0.5) & small_ok, scores, -1e9)
    box_ref[0] = jnp.concatenate([x1, y1, x2, y2], axis=0)


def _run_nms(scores3, anc3, breg3, anc3t, breg3t):
    return pl.pallas_call(
        _nms_body,
        grid=(_NIMG,),
        in_specs=[
            pl.BlockSpec((1, 1, _PAD), lambda i: (i, 0, 0)),
            pl.BlockSpec((1, 4, _PAD), lambda i: (i, 0, 0)),
            pl.BlockSpec((1, 4, _PAD), lambda i: (i, 0, 0)),
            pl.BlockSpec((1, 4, _CHUNK, _NCHUNK), lambda i: (i, 0, 0, 0)),
            pl.BlockSpec((1, 4, _CHUNK, _NCHUNK), lambda i: (i, 0, 0, 0)),
        ],
        out_specs=[
            pl.BlockSpec((1, 1, _PAD), lambda i: (i, 0, 0)),
            pl.BlockSpec((1, 4, _PAD), lambda i: (i, 0, 0)),
        ],
        out_shape=[
            jax.ShapeDtypeStruct((_NIMG, 1, _PAD), jnp.float32),
            jax.ShapeDtypeStruct((_NIMG, 4, _PAD), jnp.float32),
        ],
    )(scores3, anc3, breg3, anc3t, breg3t)


@jax.jit
def kernel(anchors, objectness, box_regression):
    n = objectness.shape[0]
    obj = jnp.transpose(objectness, (0, 2, 3, 1)).reshape(n, -1)
    obj = jax.nn.sigmoid(obj)
    breg = (
        box_regression.reshape(n, _A, 4, _H, _W)
        .transpose(0, 3, 4, 1, 2)
        .reshape(n, -1, 4)
    )
    topk_scores, topk_idx = jax.lax.top_k(obj, _PRE)
    breg_sel = jnp.take_along_axis(breg, topk_idx[:, :, None], axis=1)
    anc_sel = jnp.take_along_axis(anchors, topk_idx[:, :, None], axis=1)

    pad = _PAD - _PRE
    scores3 = jnp.pad(topk_scores, ((0, 0), (0, pad)), constant_values=-1e9)
    scores3 = scores3.reshape(n, 1, _PAD)
    anc_p = jnp.pad(anc_sel, ((0, 0), (0, pad), (0, 0)))
    breg_p = jnp.pad(breg_sel, ((0, 0), (0, pad), (0, 0)))
    anc3 = jnp.transpose(anc_p, (0, 2, 1))
    breg3 = jnp.transpose(breg_p, (0, 2, 1))
    anc3t = jnp.transpose(
        anc_p.reshape(n, _NCHUNK, _CHUNK, 4), (0, 3, 2, 1)
    )
    breg3t = jnp.transpose(
        breg_p.reshape(n, _NCHUNK, _CHUNK, 4), (0, 3, 2, 1)
    )

    sm, boxes_t = _run_nms(scores3, anc3, breg3, anc3t, breg3t)
    sm = sm.reshape(n, _PAD)[:, :_PRE]
    boxes = jnp.transpose(boxes_t, (0, 2, 1))[:, :_PRE, :]

    top_scores, top_idx = jax.lax.top_k(sm, _POST)
    out_boxes = jnp.take_along_axis(boxes, top_idx[:, :, None], axis=1)
    valid = top_scores > -1e8
    out_boxes = jnp.where(valid[:, :, None], out_boxes, 0.0)
    out_scores = jnp.where(valid, top_scores, 0.0)
    return out_boxes, out_scores


# speedup vs baseline: 3.9650x; 1.1971x over previous
import functools
import math

import jax
import jax.numpy as jnp
from jax.experimental import pallas as pl

_NIMG = 4
_A = 3
_H = 192
_W = 256
_PRE = 2000
_PAD = 2048
_CHUNK = 128
_NCHUNK = _PAD // _CHUNK
_POST = 1000
_NMS_T = 0.7
_MIN_SIZE = 0.0
_IM_W = 1024.0
_IM_H = 800.0
_CLIP = math.log(1000.0 / 16.0)


def _decode(anc, rel):
    ax1, ay1, ax2, ay2 = anc[0], anc[1], anc[2], anc[3]
    aw = ax2 - ax1 + 1.0
    ah = ay2 - ay1 + 1.0
    acx = ax1 + 0.5 * aw
    acy = ay1 + 0.5 * ah
    dx = rel[0]
    dy = rel[1]
    dw = jnp.minimum(rel[2], _CLIP)
    dh = jnp.minimum(rel[3], _CLIP)
    pcx = dx * aw + acx
    pcy = dy * ah + acy
    pw = jnp.exp(dw) * aw
    ph = jnp.exp(dh) * ah
    x1 = jnp.clip(pcx - 0.5 * pw, 0.0, _IM_W - 1.0)
    y1 = jnp.clip(pcy - 0.5 * ph, 0.0, _IM_H - 1.0)
    x2 = jnp.clip(pcx + 0.5 * pw - 1.0, 0.0, _IM_W - 1.0)
    y2 = jnp.clip(pcy + 0.5 * ph - 1.0, 0.0, _IM_H - 1.0)
    return x1, y1, x2, y2


def _nms_body(s_ref, anc_ref, breg_ref, anct_ref, bregt_ref, sm_ref, box_ref):
    scores = s_ref[0]
    anc = anc_ref[0]
    rel = breg_ref[0]
    x1, y1, x2, y2 = _decode(
        [anc[i : i + 1, :] for i in range(4)],
        [rel[i : i + 1, :] for i in range(4)],
    )
    anct = anct_ref[0]
    relt = bregt_ref[0]
    x1t, y1t, x2t, y2t = _decode(
        [anct[i] for i in range(4)],
        [relt[i] for i in range(4)],
    )

    ws = x2 - x1 + 1.0
    hs = y2 - y1 + 1.0
    small_ok = (ws >= _MIN_SIZE) & (hs >= _MIN_SIZE)
    area = ws * hs
    wst = x2t - x1t + 1.0
    hst = y2t - y1t + 1.0
    areat = wst * hst

    lane128 = jax.lax.broadcasted_iota(jnp.int32, (1, _CHUNK), 1)
    glob = jax.lax.broadcasted_iota(jnp.int32, (1, _PAD), 1)
    tmat = (
        jax.lax.broadcasted_iota(jnp.int32, (_CHUNK, _CHUNK), 0)
        == jax.lax.broadcasted_iota(jnp.int32, (_CHUNK, _CHUNK), 1)
    )

    keep = jnp.ones((1, _PAD), jnp.float32)

    for c in range(_NCHUNK):
        lo = c * _CHUNK
        hi = lo + _CHUNK
        cx1 = x1[:, lo:hi]
        cy1 = y1[:, lo:hi]
        cx2 = x2[:, lo:hi]
        cy2 = y2[:, lo:hi]
        car = area[:, lo:hi]
        kchunk = keep[:, lo:hi]

        def body(k, kc, cx1=cx1, cy1=cy1, cx2=cx2, cy2=cy2, car=car):
            sel = lane128 == k
            kx1 = jnp.max(jnp.where(sel, cx1, -1e30), keepdims=True)
            ky1 = jnp.max(jnp.where(sel, cy1, -1e30), keepdims=True)
            kx2 = jnp.max(jnp.where(sel, cx2, -1e30), keepdims=True)
            ky2 = jnp.max(jnp.where(sel, cy2, -1e30), keepdims=True)
            kar = jnp.max(jnp.where(sel, car, -1e30), keepdims=True)
            klive = jnp.max(jnp.where(sel, kc, 0.0), keepdims=True)
            iw = jnp.maximum(jnp.minimum(cx2, kx2) - jnp.maximum(cx1, kx1) + 1.0, 0.0)
            ih = jnp.maximum(jnp.minimum(cy2, ky2) - jnp.maximum(cy1, ky1) + 1.0, 0.0)
            inter = iw * ih
            iou = inter / (car + kar - inter)
            sup = (iou > _NMS_T) & (lane128 > k) & (klive > 0.5)
            return kc * jnp.where(sup, 0.0, 1.0)

        kchunk = jax.lax.fori_loop(0, _CHUNK, body, kchunk)

        kt = jnp.max(
            jnp.where(tmat & (kchunk > 0.5), 1.0, 0.0), axis=1, keepdims=True
        )
        ccx1 = x1t[:, c : c + 1]
        ccy1 = y1t[:, c : c + 1]
        ccx2 = x2t[:, c : c + 1]
        ccy2 = y2t[:, c : c + 1]
        ccar = areat[:, c : c + 1]

        iw = jnp.maximum(jnp.minimum(x2, ccx2) - jnp.maximum(x1, ccx1) + 1.0, 0.0)
        ih = jnp.maximum(jnp.minimum(y2, ccy2) - jnp.maximum(y1, ccy1) + 1.0, 0.0)
        inter = iw * ih
        iou = inter / (area + ccar - inter)
        supm = (iou > _NMS_T) & (glob >= hi) & (kt > 0.5)
        sup_any = jnp.max(jnp.where(supm, 1.0, 0.0), axis=0, keepdims=True)

        parts = []
        if lo > 0:
            parts.append(keep[:, :lo])
        parts.append(kchunk)
        if hi < _PAD:
            parts.append(keep[:, hi:])
        keep = jnp.concatenate(parts, axis=1) * jnp.where(sup_any > 0.5, 0.0, 1.0)

    sm_ref[0] = jnp.where((keep > 0.5) & small_ok, scores, -1e9)
    box_ref[0] = jnp.concatenate([x1, y1, x2, y2], axis=0)


def _run_nms(scores3, anc3, breg3, anc3t, breg3t):
    return pl.pallas_call(
        _nms_body,
        grid=(_NIMG,),
        in_specs=[
            pl.BlockSpec((1, 1, _PAD), lambda i: (i, 0, 0)),
            pl.BlockSpec((1, 4, _PAD), lambda i: (i, 0, 0)),
            pl.BlockSpec((1, 4, _PAD), lambda i: (i, 0, 0)),
            pl.BlockSpec((1, 4, _CHUNK, _NCHUNK), lambda i: (i, 0, 0, 0)),
            pl.BlockSpec((1, 4, _CHUNK, _NCHUNK), lambda i: (i, 0, 0, 0)),
        ],
        out_specs=[
            pl.BlockSpec((1, 1, _PAD), lambda i: (i, 0, 0)),
            pl.BlockSpec((1, 4, _PAD), lambda i: (i, 0, 0)),
        ],
        out_shape=[
            jax.ShapeDtypeStruct((_NIMG, 1, _PAD), jnp.float32),
            jax.ShapeDtypeStruct((_NIMG, 4, _PAD), jnp.float32),
        ],
    )(scores3, anc3, breg3, anc3t, breg3t)


@jax.jit
def kernel(anchors, objectness, box_regression):
    n = objectness.shape[0]
    obj = jnp.transpose(objectness, (0, 2, 3, 1)).reshape(n, -1)
    obj = jax.nn.sigmoid(obj)
    topk_scores, topk_idx = jax.lax.top_k(obj, _PRE)
    a_idx = topk_idx % _A
    hw_idx = topk_idx // _A
    flat4 = (a_idx[:, :, None] * 4 + jnp.arange(4)[None, None, :]) * (_H * _W) + hw_idx[
        :, :, None
    ]
    breg_sel = jnp.take_along_axis(
        box_regression.reshape(n, -1), flat4.reshape(n, -1), axis=1
    ).reshape(n, _PRE, 4)
    anc_sel = jnp.take_along_axis(anchors, topk_idx[:, :, None], axis=1)

    pad = _PAD - _PRE
    scores3 = jnp.pad(topk_scores, ((0, 0), (0, pad)), constant_values=-1e9)
    scores3 = scores3.reshape(n, 1, _PAD)
    anc_p = jnp.pad(anc_sel, ((0, 0), (0, pad), (0, 0)))
    breg_p = jnp.pad(breg_sel, ((0, 0), (0, pad), (0, 0)))
    anc3 = jnp.transpose(anc_p, (0, 2, 1))
    breg3 = jnp.transpose(breg_p, (0, 2, 1))
    anc3t = jnp.transpose(
        anc_p.reshape(n, _NCHUNK, _CHUNK, 4), (0, 3, 2, 1)
    )
    breg3t = jnp.transpose(
        breg_p.reshape(n, _NCHUNK, _CHUNK, 4), (0, 3, 2, 1)
    )

    sm, boxes_t = _run_nms(scores3, anc3, breg3, anc3t, breg3t)
    sm = sm.reshape(n, _PAD)[:, :_PRE]
    boxes = jnp.transpose(boxes_t, (0, 2, 1))[:, :_PRE, :]

    top_scores, top_idx = jax.lax.top_k(sm, _POST)
    out_boxes = jnp.take_along_axis(boxes, top_idx[:, :, None], axis=1)
    valid = top_scores > -1e8
    out_boxes = jnp.where(valid[:, :, None], out_boxes, 0.0)
    out_scores = jnp.where(valid, top_scores, 0.0)
    return out_boxes, out_scores


# speedup vs baseline: 4.7951x; 1.2094x over previous
import functools
import math

import jax
import jax.numpy as jnp
from jax.experimental import pallas as pl

_NIMG = 4
_A = 3
_H = 192
_W = 256
_PRE = 2000
_PAD = 2048
_CHUNK = 128
_NCHUNK = _PAD // _CHUNK
_POST = 1000
_NMS_T = 0.7
_MIN_SIZE = 0.0
_IM_W = 1024.0
_IM_H = 800.0
_CLIP = math.log(1000.0 / 16.0)


def _decode(anc, rel):
    ax1, ay1, ax2, ay2 = anc[0], anc[1], anc[2], anc[3]
    aw = ax2 - ax1 + 1.0
    ah = ay2 - ay1 + 1.0
    acx = ax1 + 0.5 * aw
    acy = ay1 + 0.5 * ah
    dx = rel[0]
    dy = rel[1]
    dw = jnp.minimum(rel[2], _CLIP)
    dh = jnp.minimum(rel[3], _CLIP)
    pcx = dx * aw + acx
    pcy = dy * ah + acy
    pw = jnp.exp(dw) * aw
    ph = jnp.exp(dh) * ah
    x1 = jnp.clip(pcx - 0.5 * pw, 0.0, _IM_W - 1.0)
    y1 = jnp.clip(pcy - 0.5 * ph, 0.0, _IM_H - 1.0)
    x2 = jnp.clip(pcx + 0.5 * pw - 1.0, 0.0, _IM_W - 1.0)
    y2 = jnp.clip(pcy + 0.5 * ph - 1.0, 0.0, _IM_H - 1.0)
    return x1, y1, x2, y2


def _nms_body(s_ref, anc_ref, breg_ref, anct_ref, bregt_ref, sm_ref, box_ref):
    scores = s_ref[0]
    anc = anc_ref[0]
    rel = breg_ref[0]
    x1, y1, x2, y2 = _decode(
        [anc[i : i + 1, :] for i in range(4)],
        [rel[i : i + 1, :] for i in range(4)],
    )
    anct = anct_ref[0]
    relt = bregt_ref[0]
    x1t, y1t, x2t, y2t = _decode(
        [anct[i] for i in range(4)],
        [relt[i] for i in range(4)],
    )

    ws = x2 - x1 + 1.0
    hs = y2 - y1 + 1.0
    small_ok = (ws >= _MIN_SIZE) & (hs >= _MIN_SIZE)
    area = ws * hs
    wst = x2t - x1t + 1.0
    hst = y2t - y1t + 1.0
    areat = wst * hst

    lane128 = jax.lax.broadcasted_iota(jnp.int32, (1, _CHUNK), 1)
    glob = jax.lax.broadcasted_iota(jnp.int32, (1, _PAD), 1)
    tmat = (
        jax.lax.broadcasted_iota(jnp.int32, (_CHUNK, _CHUNK), 0)
        == jax.lax.broadcasted_iota(jnp.int32, (_CHUNK, _CHUNK), 1)
    )

    keep = jnp.ones((1, _PAD), jnp.float32)

    for c in range(_NCHUNK):
        lo = c * _CHUNK
        hi = lo + _CHUNK
        cx1 = x1[:, lo:hi]
        cy1 = y1[:, lo:hi]
        cx2 = x2[:, lo:hi]
        cy2 = y2[:, lo:hi]
        car = area[:, lo:hi]
        kchunk = keep[:, lo:hi]

        def body(k, kc, cx1=cx1, cy1=cy1, cx2=cx2, cy2=cy2, car=car):
            sel = lane128 == k
            kx1 = jnp.max(jnp.where(sel, cx1, -1e30), keepdims=True)
            ky1 = jnp.max(jnp.where(sel, cy1, -1e30), keepdims=True)
            kx2 = jnp.max(jnp.where(sel, cx2, -1e30), keepdims=True)
            ky2 = jnp.max(jnp.where(sel, cy2, -1e30), keepdims=True)
            kar = jnp.max(jnp.where(sel, car, -1e30), keepdims=True)
            klive = jnp.max(jnp.where(sel, kc, 0.0), keepdims=True)
            iw = jnp.maximum(jnp.minimum(cx2, kx2) - jnp.maximum(cx1, kx1) + 1.0, 0.0)
            ih = jnp.maximum(jnp.minimum(cy2, ky2) - jnp.maximum(cy1, ky1) + 1.0, 0.0)
            inter = iw * ih
            iou = inter / (car + kar - inter)
            sup = (iou > _NMS_T) & (lane128 > k) & (klive > 0.5)
            return kc * jnp.where(sup, 0.0, 1.0)

        kchunk = jax.lax.fori_loop(0, _CHUNK, body, kchunk)

        kt = jnp.max(
            jnp.where(tmat & (kchunk > 0.5), 1.0, 0.0), axis=1, keepdims=True
        )
        ccx1 = x1t[:, c : c + 1]
        ccy1 = y1t[:, c : c + 1]
        ccx2 = x2t[:, c : c + 1]
        ccy2 = y2t[:, c : c + 1]
        ccar = areat[:, c : c + 1]

        iw = jnp.maximum(jnp.minimum(x2, ccx2) - jnp.maximum(x1, ccx1) + 1.0, 0.0)
        ih = jnp.maximum(jnp.minimum(y2, ccy2) - jnp.maximum(y1, ccy1) + 1.0, 0.0)
        inter = iw * ih
        iou = inter / (area + ccar - inter)
        supm = (iou > _NMS_T) & (glob >= hi) & (kt > 0.5)
        sup_any = jnp.max(jnp.where(supm, 1.0, 0.0), axis=0, keepdims=True)

        parts = []
        if lo > 0:
            parts.append(keep[:, :lo])
        parts.append(kchunk)
        if hi < _PAD:
            parts.append(keep[:, hi:])
        keep = jnp.concatenate(parts, axis=1) * jnp.where(sup_any > 0.5, 0.0, 1.0)

    sm_ref[0] = jnp.where((keep > 0.5) & small_ok, scores, -1e9)
    box_ref[0] = jnp.concatenate([x1, y1, x2, y2], axis=0)


def _run_nms(scores3, anc3, breg3, anc3t, breg3t):
    return pl.pallas_call(
        _nms_body,
        grid=(_NIMG,),
        in_specs=[
            pl.BlockSpec((1, 1, _PAD), lambda i: (i, 0, 0)),
            pl.BlockSpec((1, 4, _PAD), lambda i: (i, 0, 0)),
            pl.BlockSpec((1, 4, _PAD), lambda i: (i, 0, 0)),
            pl.BlockSpec((1, 4, _CHUNK, _NCHUNK), lambda i: (i, 0, 0, 0)),
            pl.BlockSpec((1, 4, _CHUNK, _NCHUNK), lambda i: (i, 0, 0, 0)),
        ],
        out_specs=[
            pl.BlockSpec((1, 1, _PAD), lambda i: (i, 0, 0)),
            pl.BlockSpec((1, 4, _PAD), lambda i: (i, 0, 0)),
        ],
        out_shape=[
            jax.ShapeDtypeStruct((_NIMG, 1, _PAD), jnp.float32),
            jax.ShapeDtypeStruct((_NIMG, 4, _PAD), jnp.float32),
        ],
    )(scores3, anc3, breg3, anc3t, breg3t)


@jax.jit
def kernel(anchors, objectness, box_regression):
    n = objectness.shape[0]
    s_raw = jax.nn.sigmoid(objectness.reshape(n, _A, _H * _W))
    ch_scores, ch_hw = jax.lax.top_k(s_raw, _PRE)
    ch_flat = ch_hw * _A + jnp.arange(_A, dtype=ch_hw.dtype)[None, :, None]
    cand_scores = ch_scores.reshape(n, _A * _PRE)
    cand_idx = ch_flat.reshape(n, _A * _PRE)
    neg_sorted, idx_sorted = jax.lax.sort(
        (-cand_scores, cand_idx), dimension=1, num_keys=2
    )
    topk_scores = -neg_sorted[:, :_PRE]
    topk_idx = idx_sorted[:, :_PRE]
    a_idx = topk_idx % _A
    hw_idx = topk_idx // _A
    flat4 = (a_idx[:, :, None] * 4 + jnp.arange(4)[None, None, :]) * (_H * _W) + hw_idx[
        :, :, None
    ]
    breg_sel = jnp.take_along_axis(
        box_regression.reshape(n, -1), flat4.reshape(n, -1), axis=1
    ).reshape(n, _PRE, 4)
    anc_sel = jnp.take_along_axis(anchors, topk_idx[:, :, None], axis=1)

    pad = _PAD - _PRE
    scores3 = jnp.pad(topk_scores, ((0, 0), (0, pad)), constant_values=-1e9)
    scores3 = scores3.reshape(n, 1, _PAD)
    anc_p = jnp.pad(anc_sel, ((0, 0), (0, pad), (0, 0)))
    breg_p = jnp.pad(breg_sel, ((0, 0), (0, pad), (0, 0)))
    anc3 = jnp.transpose(anc_p, (0, 2, 1))
    breg3 = jnp.transpose(breg_p, (0, 2, 1))
    anc3t = jnp.transpose(
        anc_p.reshape(n, _NCHUNK, _CHUNK, 4), (0, 3, 2, 1)
    )
    breg3t = jnp.transpose(
        breg_p.reshape(n, _NCHUNK, _CHUNK, 4), (0, 3, 2, 1)
    )

    sm, boxes_t = _run_nms(scores3, anc3, breg3, anc3t, breg3t)
    sm = sm.reshape(n, _PAD)[:, :_PRE]
    boxes = jnp.transpose(boxes_t, (0, 2, 1))[:, :_PRE, :]

    top_scores, top_idx = jax.lax.top_k(sm, _POST)
    out_boxes = jnp.take_along_axis(boxes, top_idx[:, :, None], axis=1)
    valid = top_scores > -1e8
    out_boxes = jnp.where(valid[:, :, None], out_boxes, 0.0)
    out_scores = jnp.where(valid, top_scores, 0.0)
    return out_boxes, out_scores
